# baseline (speedup 1.0000x reference)
import functools

import jax
import jax.numpy as jnp
from jax import lax
from jax.experimental import pallas as pl
from jax.experimental.pallas import tpu as pltpu
from jax.experimental.pallas import tpu_sc as plsc

NC = 2
NS = 16
NW = NC * NS

SL = 128
KW = 2
C = KW * SL


def _make_kernel(n_rows, pair_dim):
    per_w = n_rows // NW
    n_chunks = per_w // C
    n2 = n_chunks // 2
    assert n_chunks % 2 == 0 and n2 >= 2

    mesh = plsc.VectorSubcoreMesh(core_axis_name="c", subcore_axis_name="s",
                                  num_cores=NC, num_subcores=NS)

    idx_t = pltpu.VMEM((KW, SL), jnp.int32)
    big_t = pltpu.VMEM((C, pair_dim), jnp.float32)

    @functools.partial(
        pl.kernel,
        out_type=jax.ShapeDtypeStruct((n_rows, pair_dim), jnp.float32),
        mesh=mesh,
        scratch_types=[
            idx_t, idx_t, big_t, big_t,
            pltpu.SemaphoreType.DMA, pltpu.SemaphoreType.DMA,
            pltpu.SemaphoreType.DMA, pltpu.SemaphoreType.DMA,
        ],
    )
    def k(idxall, wtab2, out, idxA, idxB, bigA, bigB, gsA, gsB, wsA, wsB):
        wid = lax.axis_index("s") * NC + lax.axis_index("c")

        def gather_descs(idxv, big, gsem):
            return [
                pltpu.make_async_copy(
                    wtab2.at[idxv.at[t]], big.at[pl.ds(t * SL, SL)], gsem)
                for t in range(KW)
            ]

        def start(j, idxv, big, gsem):
            pltpu.sync_copy(idxall.at[wid, j], idxv)
            for d in gather_descs(idxv, big, gsem):
                d.start()

        def wait_gathers(idxv, big, gsem):
            for d in gather_descs(idxv, big, gsem):
                d.wait()

        def write_desc(j, big, wsem):
            base = wid * per_w + j * C
            return pltpu.make_async_copy(big, out.at[pl.ds(base, C), :],
                                         wsem)

        def wait_write(big, wsem):
            write_desc(0, big, wsem).wait()

        A = (idxA, bigA, gsA)
        B = (idxB, bigB, gsB)

        def pair_step(i, first=False, last=False):
            j = 2 * i
            wait_gathers(*A)
            start(j + 1, *B)
            if not first:
                wait_write(bigA, wsA)
            write_desc(j, bigA, wsA).start()
            wait_gathers(*B)
            if not last:
                start(j + 2, *A)
            if not first:
                wait_write(bigB, wsB)
            write_desc(j + 1, bigB, wsB).start()

        start(0, *A)
        pair_step(0, first=True)

        @pl.loop(1, n2 - 1)
        def body(i):
            pair_step(i)

        pair_step(n2 - 1, last=True)
        wait_write(bigA, wsA)
        wait_write(bigB, wsB)

    return k


PSL = 128
PKW = 4
PC = PKW * PSL


def _make_pos_kernel(n_rows, pos_dim):
    per_w = n_rows // NW
    n_chunks = per_w // PC
    n2 = n_chunks // 2
    assert n_chunks % 2 == 0 and n2 >= 2

    mesh = plsc.VectorSubcoreMesh(core_axis_name="c", subcore_axis_name="s",
                                  num_cores=NC, num_subcores=NS)

    idx_t = pltpu.VMEM((2 * PKW, PSL), jnp.int32)
    buf_t = pltpu.VMEM((PC, pos_dim), jnp.float32)

    @functools.partial(
        pl.kernel,
        out_type=jax.ShapeDtypeStruct((n_rows, 2 * pos_dim), jnp.float32),
        mesh=mesh,
        compiler_params=pltpu.CompilerParams(use_tc_tiling_on_sc=False),
        scratch_types=[
            idx_t, idx_t, buf_t, buf_t, buf_t, buf_t,
            pltpu.SemaphoreType.DMA, pltpu.SemaphoreType.DMA,
            pltpu.SemaphoreType.DMA, pltpu.SemaphoreType.DMA,
        ],
    )
    def k(pidx, p1tab, p2tab, out,
          idxA, idxB, b1A, b1B, b2A, b2B, gsA, gsB, wsA, wsB):
        wid = lax.axis_index("s") * NC + lax.axis_index("c")

        def gather_descs(idxv, b1, b2, gsem):
            ds = []
            for t in range(PKW):
                rows = pl.ds(t * PSL, PSL)
                ds.append(pltpu.make_async_copy(
                    p1tab.at[idxv.at[t]], b1.at[rows], gsem))
                ds.append(pltpu.make_async_copy(
                    p2tab.at[idxv.at[PKW + t]], b2.at[rows], gsem))
            return ds

        def start(j, idxv, b1, b2, gsem):
            pltpu.sync_copy(pidx.at[wid, j], idxv)
            for d in gather_descs(idxv, b1, b2, gsem):
                d.start()

        def wait_gathers(idxv, b1, b2, gsem):
            for d in gather_descs(idxv, b1, b2, gsem):
                d.wait()

        def write_descs(j, b1, b2, wsem):
            base = wid * per_w + j * PC
            return [
                pltpu.make_async_copy(
                    b1, out.at[pl.ds(base, PC), pl.ds(0, pos_dim)], wsem),
                pltpu.make_async_copy(
                    b2, out.at[pl.ds(base, PC), pl.ds(pos_dim, pos_dim)],
                    wsem),
            ]

        def wait_writes(b1, b2, wsem):
            for d in write_descs(0, b1, b2, wsem):
                d.wait()

        A = (idxA, b1A, b2A, gsA)
        B = (idxB, b1B, b2B, gsB)

        def pair_step(i, first=False, last=False):
            j = 2 * i
            wait_gathers(*A)
            start(j + 1, *B)
            if not first:
                wait_writes(b1A, b2A, wsA)
            for d in write_descs(j, b1A, b2A, wsA):
                d.start()
            wait_gathers(*B)
            if not last:
                start(j + 2, *A)
            if not first:
                wait_writes(b1B, b2B, wsB)
            for d in write_descs(j + 1, b1B, b2B, wsB):
                d.start()

        start(0, *A)
        pair_step(0, first=True)

        @pl.loop(1, n2 - 1)
        def body(i):
            pair_step(i)

        pair_step(n2 - 1, last=True)
        wait_writes(b1A, b2A, wsA)
        wait_writes(b1B, b2B, wsB)

    return k


def kernel(word, pos1, pos2, word_table, pos1_table, pos2_table):
    b, s = word.shape
    vocab, word_dim = word_table.shape
    pos_dim = pos1_table.shape[1]
    out_dim = word_dim + 2 * pos_dim
    n = b * s
    per_w = n // NW
    n_chunks = per_w // C

    wtab2 = word_table.reshape(vocab // 2, 2 * word_dim)

    wordf = word.reshape(-1)
    idxall = (wordf >> 1).reshape(NW, n_chunks, KW, SL)

    k = _make_kernel(n, 2 * word_dim)
    pairs = k(idxall, wtab2)

    n_chunks_p = per_w // PC
    pidx = jnp.stack(
        [pos1.reshape(NW, n_chunks_p, PKW, PSL),
         pos2.reshape(NW, n_chunks_p, PKW, PSL)],
        axis=2,
    ).reshape(NW, n_chunks_p, 2 * PKW, PSL)

    kp = _make_pos_kernel(n, pos_dim)
    pose = kp(pidx, pos1_table, pos2_table)

    parity = (wordf & 1).astype(jnp.bool_)
    wpart = jnp.where(parity[:, None], pairs[:, word_dim:], pairs[:, :word_dim])
    out = jnp.concatenate([wpart, pose], axis=1)
    return out.reshape(b, s, out_dim)


# speedup vs baseline: 1.0006x; 1.0006x over previous
import functools

import jax
import jax.numpy as jnp
from jax import lax
from jax.experimental import pallas as pl
from jax.experimental.pallas import tpu as pltpu
from jax.experimental.pallas import tpu_sc as plsc

NC = 2
NS = 16
NW = NC * NS

SL = 128
KW = 2
C = KW * SL


def _make_kernel(n_rows, pair_dim):
    per_w = n_rows // NW
    n_chunks = per_w // C
    n2 = n_chunks // 2
    assert n_chunks % 2 == 0 and n2 >= 2

    mesh = plsc.VectorSubcoreMesh(core_axis_name="c", subcore_axis_name="s",
                                  num_cores=NC, num_subcores=NS)

    idx_t = pltpu.VMEM((KW, SL), jnp.int32)
    big_t = pltpu.VMEM((C, pair_dim), jnp.float32)

    @functools.partial(
        pl.kernel,
        out_type=jax.ShapeDtypeStruct((n_rows, pair_dim), jnp.float32),
        mesh=mesh,
        scratch_types=[
            idx_t, idx_t, big_t, big_t,
            pltpu.SemaphoreType.DMA, pltpu.SemaphoreType.DMA,
            pltpu.SemaphoreType.DMA, pltpu.SemaphoreType.DMA,
        ],
    )
    def k(idxall, wtab2, out, idxA, idxB, bigA, bigB, gsA, gsB, wsA, wsB):
        wid = lax.axis_index("s") * NC + lax.axis_index("c")

        def gather_descs(idxv, big, gsem):
            return [
                pltpu.make_async_copy(
                    wtab2.at[idxv.at[t]], big.at[pl.ds(t * SL, SL)], gsem)
                for t in range(KW)
            ]

        def start(j, idxv, big, gsem):
            pltpu.sync_copy(idxall.at[wid, j], idxv)
            for d in gather_descs(idxv, big, gsem):
                d.start()

        def wait_gathers(idxv, big, gsem):
            for d in gather_descs(idxv, big, gsem):
                d.wait()

        def write_desc(j, big, wsem):
            base = wid * per_w + j * C
            return pltpu.make_async_copy(big, out.at[pl.ds(base, C), :],
                                         wsem)

        def wait_write(big, wsem):
            write_desc(0, big, wsem).wait()

        A = (idxA, bigA, gsA)
        B = (idxB, bigB, gsB)

        def pair_step(i, first=False, last=False):
            j = 2 * i
            wait_gathers(*A)
            if not first:
                wait_write(bigB, wsB)
            start(j + 1, *B)
            write_desc(j, bigA, wsA).start()
            wait_gathers(*B)
            wait_write(bigA, wsA)
            if not last:
                start(j + 2, *A)
            write_desc(j + 1, bigB, wsB).start()

        start(0, *A)
        pair_step(0, first=True)

        @pl.loop(1, n2 - 1)
        def body(i):
            pair_step(i)

        pair_step(n2 - 1, last=True)
        wait_write(bigB, wsB)

    return k


PSL = 128
PKW = 4
PC = PKW * PSL


def _make_pos_kernel(n_rows, pos_dim):
    per_w = n_rows // NW
    n_chunks = per_w // PC
    n2 = n_chunks // 2
    assert n_chunks % 2 == 0 and n2 >= 2

    mesh = plsc.VectorSubcoreMesh(core_axis_name="c", subcore_axis_name="s",
                                  num_cores=NC, num_subcores=NS)

    idx_t = pltpu.VMEM((2 * PKW, PSL), jnp.int32)
    buf_t = pltpu.VMEM((PC, pos_dim), jnp.float32)

    @functools.partial(
        pl.kernel,
        out_type=jax.ShapeDtypeStruct((n_rows, 2 * pos_dim), jnp.float32),
        mesh=mesh,
        compiler_params=pltpu.CompilerParams(use_tc_tiling_on_sc=False),
        scratch_types=[
            idx_t, idx_t, buf_t, buf_t, buf_t, buf_t,
            pltpu.SemaphoreType.DMA, pltpu.SemaphoreType.DMA,
            pltpu.SemaphoreType.DMA, pltpu.SemaphoreType.DMA,
        ],
    )
    def k(pidx, p1tab, p2tab, out,
          idxA, idxB, b1A, b1B, b2A, b2B, gsA, gsB, wsA, wsB):
        wid = lax.axis_index("s") * NC + lax.axis_index("c")

        def gather_descs(idxv, b1, b2, gsem):
            ds = []
            for t in range(PKW):
                rows = pl.ds(t * PSL, PSL)
                ds.append(pltpu.make_async_copy(
                    p1tab.at[idxv.at[t]], b1.at[rows], gsem))
                ds.append(pltpu.make_async_copy(
                    p2tab.at[idxv.at[PKW + t]], b2.at[rows], gsem))
            return ds

        def start(j, idxv, b1, b2, gsem):
            pltpu.sync_copy(pidx.at[wid, j], idxv)
            for d in gather_descs(idxv, b1, b2, gsem):
                d.start()

        def wait_gathers(idxv, b1, b2, gsem):
            for d in gather_descs(idxv, b1, b2, gsem):
                d.wait()

        def write_descs(j, b1, b2, wsem):
            base = wid * per_w + j * PC
            return [
                pltpu.make_async_copy(
                    b1, out.at[pl.ds(base, PC), pl.ds(0, pos_dim)], wsem),
                pltpu.make_async_copy(
                    b2, out.at[pl.ds(base, PC), pl.ds(pos_dim, pos_dim)],
                    wsem),
            ]

        def wait_writes(b1, b2, wsem):
            for d in write_descs(0, b1, b2, wsem):
                d.wait()

        A = (idxA, b1A, b2A, gsA)
        B = (idxB, b1B, b2B, gsB)

        def pair_step(i, first=False, last=False):
            j = 2 * i
            wait_gathers(*A)
            if not first:
                wait_writes(b1B, b2B, wsB)
            start(j + 1, *B)
            for d in write_descs(j, b1A, b2A, wsA):
                d.start()
            wait_gathers(*B)
            wait_writes(b1A, b2A, wsA)
            if not last:
                start(j + 2, *A)
            for d in write_descs(j + 1, b1B, b2B, wsB):
                d.start()

        start(0, *A)
        pair_step(0, first=True)

        @pl.loop(1, n2 - 1)
        def body(i):
            pair_step(i)

        pair_step(n2 - 1, last=True)
        wait_writes(b1B, b2B, wsB)

    return k


def kernel(word, pos1, pos2, word_table, pos1_table, pos2_table):
    b, s = word.shape
    vocab, word_dim = word_table.shape
    pos_dim = pos1_table.shape[1]
    out_dim = word_dim + 2 * pos_dim
    n = b * s
    per_w = n // NW
    n_chunks = per_w // C

    wtab2 = word_table.reshape(vocab // 2, 2 * word_dim)

    wordf = word.reshape(-1)
    idxall = (wordf >> 1).reshape(NW, n_chunks, KW, SL)

    k = _make_kernel(n, 2 * word_dim)
    pairs = k(idxall, wtab2)

    n_chunks_p = per_w // PC
    pidx = jnp.stack(
        [pos1.reshape(NW, n_chunks_p, PKW, PSL),
         pos2.reshape(NW, n_chunks_p, PKW, PSL)],
        axis=2,
    ).reshape(NW, n_chunks_p, 2 * PKW, PSL)

    kp = _make_pos_kernel(n, pos_dim)
    pose = kp(pidx, pos1_table, pos2_table)

    parity = (wordf & 1).astype(jnp.bool_)
    wpart = jnp.where(parity[:, None], pairs[:, word_dim:], pairs[:, :word_dim])
    out = jnp.concatenate([wpart, pose], axis=1)
    return out.reshape(b, s, out_dim)


# speedup vs baseline: 1.8049x; 1.8038x over previous
import functools

import jax
import jax.numpy as jnp
from jax import lax
from jax.experimental import pallas as pl
from jax.experimental.pallas import tpu as pltpu
from jax.experimental.pallas import tpu_sc as plsc

NC = 2
NS = 16
NW = NC * NS

SL = 128
KW = 4
C = KW * SL


def _make_kernel(n_rows, word_dim, pos_dim, out_dim):
    per_w = n_rows // NW
    n_chunks = per_w // C
    n2 = n_chunks // 2
    assert n_chunks % 2 == 0 and n2 >= 2
    mesh = plsc.VectorSubcoreMesh(core_axis_name="c", subcore_axis_name="s",
                                  num_cores=NC, num_subcores=NS)

    idx_t = pltpu.VMEM((3 * KW, SL), jnp.int32)
    wbuf_t = pltpu.VMEM((C, word_dim), jnp.float32)
    pbuf_t = pltpu.VMEM((C, pos_dim), jnp.float32)

    @functools.partial(
        pl.kernel,
        out_type=jax.ShapeDtypeStruct((n_rows, out_dim), jnp.float32),
        mesh=mesh,
        compiler_params=pltpu.CompilerParams(use_tc_tiling_on_sc=False),
        scratch_types=[
            idx_t, idx_t, wbuf_t, wbuf_t, pbuf_t, pbuf_t, pbuf_t, pbuf_t,
            pltpu.SemaphoreType.DMA, pltpu.SemaphoreType.DMA,
            pltpu.SemaphoreType.DMA, pltpu.SemaphoreType.DMA,
        ],
    )
    def k(idxall, wtab, p1tab, p2tab, out,
          idxA, idxB, wA, wB, p1A, p1B, p2A, p2B, gsA, gsB, wsA, wsB):
        wid = lax.axis_index("s") * NC + lax.axis_index("c")

        def gather_descs(idxv, wb, p1b, p2b, gsem):
            ds = []
            for t in range(KW):
                ds.append(pltpu.make_async_copy(
                    wtab.at[idxv.at[t]], wb.at[pl.ds(t * SL, SL)], gsem))
                ds.append(pltpu.make_async_copy(
                    p1tab.at[idxv.at[KW + t]],
                    p1b.at[pl.ds(t * SL, SL)], gsem))
                ds.append(pltpu.make_async_copy(
                    p2tab.at[idxv.at[2 * KW + t]],
                    p2b.at[pl.ds(t * SL, SL)], gsem))
            return ds

        def start(j, idxv, wb, p1b, p2b, gsem):
            pltpu.sync_copy(idxall.at[wid, j], idxv)
            for d in gather_descs(idxv, wb, p1b, p2b, gsem):
                d.start()

        def wait_gathers(idxv, wb, p1b, p2b, gsem):
            for d in gather_descs(idxv, wb, p1b, p2b, gsem):
                d.wait()

        def write_descs(j, wb, p1b, p2b, wsem):
            base = wid * per_w + j * C
            return [
                pltpu.make_async_copy(
                    wb, out.at[pl.ds(base, C), pl.ds(0, word_dim)], wsem),
                pltpu.make_async_copy(
                    p1b, out.at[pl.ds(base, C), pl.ds(word_dim, pos_dim)],
                    wsem),
                pltpu.make_async_copy(
                    p2b,
                    out.at[pl.ds(base, C), pl.ds(word_dim + pos_dim, pos_dim)],
                    wsem),
            ]

        def issue_writes(j, wb, p1b, p2b, wsem):
            for d in write_descs(j, wb, p1b, p2b, wsem):
                d.start()

        def wait_writes(wb, p1b, p2b, wsem):
            for d in write_descs(0, wb, p1b, p2b, wsem):
                d.wait()

        A = (idxA, wA, p1A, p2A, gsA)
        B = (idxB, wB, p1B, p2B, gsB)

        def half(j, cur, nxt, ws_cur, ws_nxt, first, last):
            wait_gathers(*cur)
            if not first:
                wait_writes(nxt[1], nxt[2], nxt[3], ws_nxt)
            issue_writes(j, cur[1], cur[2], cur[3], ws_cur)
            if not last:
                start(j + 1, *nxt)

        def pair(i, first=False, last=False):
            j = 2 * i
            half(j, A, B, wsA, wsB, first=first, last=False)
            half(j + 1, B, A, wsB, wsA, first=False, last=last)

        start(0, *A)
        pair(0, first=True)

        @pl.loop(1, n2 - 1)
        def body(i):
            pair(i)

        pair(n2 - 1, last=True)
        wait_writes(wB, p1B, p2B, wsB)

    return k


def kernel(word, pos1, pos2, word_table, pos1_table, pos2_table):
    b, s = word.shape
    word_dim = word_table.shape[1]
    pos_dim = pos1_table.shape[1]
    out_dim = word_dim + 2 * pos_dim
    n = b * s
    per_w = n // NW
    n_chunks = per_w // C

    def pack(a):
        return a.reshape(NW, n_chunks, KW, SL)

    idxall = jnp.stack(
        [pack(word), pack(pos1), pack(pos2)], axis=2
    ).reshape(NW, n_chunks, 3 * KW, SL)

    k = _make_kernel(n, word_dim, pos_dim, 128)
    out = k(idxall, word_table, pos1_table, pos2_table)
    return out[:, :out_dim].reshape(b, s, out_dim)
